# baseline (speedup 1.0000x reference)
import functools

import jax
import jax.numpy as jnp
from jax import lax
from jax.experimental import pallas as pl
from jax.experimental.pallas import tpu as pltpu
from jax.experimental.pallas import tpu_sc as plsc

_NC = 2
_NS = 16
_NW = _NC * _NS
_L = 16
_CHUNK = 128


def _sc_agg(n_pad, width, n_chunks):
  rows_per_sub = n_pad // _NS
  mesh = plsc.VectorSubcoreMesh(core_axis_name="c", subcore_axis_name="s")

  @functools.partial(
      pl.kernel,
      out_type=(
          jax.ShapeDtypeStruct((_NC, n_pad, width), jnp.float32),
          jax.ShapeDtypeStruct((_NW * n_pad,), jnp.float32),
      ),
      mesh=mesh,
      compiler_params=pltpu.CompilerParams(needs_layout_passes=False,
                                           disable_bounds_checks=True),
      scratch_types=[
          pltpu.VMEM((1, _CHUNK), jnp.int32),
          pltpu.VMEM((1, _CHUNK), jnp.int32),
          pltpu.VMEM((1, _CHUNK), jnp.int32),
          pltpu.VMEM((1, _CHUNK), jnp.int32),
          pltpu.VMEM((1, _CHUNK), jnp.int32),
          pltpu.VMEM((1, _CHUNK), jnp.int32),
          pltpu.VMEM((1, _CHUNK), jnp.int32),
          pltpu.VMEM((1, _CHUNK), jnp.int32),
          pltpu.VMEM((_CHUNK, width), jnp.float32),
          pltpu.VMEM((_CHUNK, width), jnp.float32),
          pltpu.VMEM((n_pad,), jnp.float32),
          pltpu.VMEM_SHARED((n_pad, width), jnp.float32),
          pltpu.SemaphoreType.DMA,
          pltpu.SemaphoreType.DMA,
          pltpu.SemaphoreType.DMA,
      ],
  )
  def k(src_hbm, dst_hbm, tab_hbm, z2_hbm,
        out_acc, out_degp, sidx0, sidx1, sidx2, sidx3,
        didx0, didx1, didx2, didx3, rows0, rows1,
        deg_v, acc_s, gsem0, gsem1, isem):
    sidx = (sidx0, sidx1, sidx2, sidx3)
    didx = (didx0, didx1, didx2, didx3)
    c = lax.axis_index("c")
    s = lax.axis_index("s")
    wid = s * _NC + c
    row0 = s * rows_per_sub

    pltpu.sync_copy(z2_hbm, acc_s.at[pl.ds(row0, rows_per_sub)])

    def zbody(i, carry):
      deg_v[pl.ds(i * _L, _L)] = jnp.zeros((_L,), jnp.float32)
      return carry

    lax.fori_loop(0, n_pad // _L, zbody, 0)
    plsc.subcore_barrier()

    ones = jnp.ones((_L,), jnp.float32)

    def idx_load(i, b):
      pltpu.async_copy(src_hbm.at[wid, i], sidx[b], isem)
      pltpu.async_copy(dst_hbm.at[wid, i], didx[b], isem)

    def idx_wait(b):
      pltpu.make_async_copy(src_hbm.at[0, 0], sidx[b], isem).wait()
      pltpu.make_async_copy(dst_hbm.at[0, 0], didx[b], isem).wait()

    def gather(b, buf, sem):
      pltpu.async_copy(tab_hbm.at[sidx[b].at[0]], buf, sem)

    def gwait(b, buf, sem):
      pltpu.make_async_copy(tab_hbm.at[sidx[b].at[0]], buf, sem).wait()

    def process(i, b, buf, sem):
      nb = (b + 2) % 4
      idx_load(i + 2, nb)
      gwait(b, buf, sem)
      for kk in range(_CHUNK // _L):
        plsc.addupdate_scatter(deg_v, [didx[b][0, pl.ds(kk * _L, _L)]], ones)
      pltpu.sync_copy(buf, acc_s.at[didx[b].at[0]], add=True)
      idx_wait(nb)
      gather(nb, buf, sem)

    idx_load(0, 0)
    idx_load(1, 1)
    idx_wait(0)
    gather(0, rows0, gsem0)
    idx_wait(1)
    gather(1, rows1, gsem1)

    def body(q, carry):
      i0 = 4 * q
      process(i0, 0, rows0, gsem0)
      process(i0 + 1, 1, rows1, gsem1)
      process(i0 + 2, 2, rows0, gsem0)
      process(i0 + 3, 3, rows1, gsem1)
      return carry

    lax.fori_loop(0, n_chunks // 4, body, 0)
    gwait(0, rows0, gsem0)
    gwait(1, rows1, gsem1)
    plsc.subcore_barrier()

    pltpu.sync_copy(acc_s.at[pl.ds(row0, rows_per_sub)],
                    out_acc.at[c, pl.ds(row0, rows_per_sub)])
    pltpu.sync_copy(deg_v, out_degp.at[pl.ds(wid * n_pad, n_pad)])

  return k


def _combine(h, a0, a1, dp, w_self, w_neigh, bias, relu):
  n, d_in = h.shape
  d_out = w_self.shape[1]
  blk = 1000
  grid = n // blk

  def body(h_r, a0_r, a1_r, dp_r, ws_r, wn_r, b_r, o_r):
    deg = jnp.sum(dp_r[...], axis=1, keepdims=True)
    r = 1.0 / jnp.maximum(deg, 1.0)
    agg = (a0_r[...] + a1_r[...]) * r
    o = (jnp.dot(h_r[...], ws_r[...], preferred_element_type=jnp.float32)
         + jnp.dot(agg, wn_r[...], preferred_element_type=jnp.float32)
         + b_r[...])
    o_r[...] = jnp.maximum(o, 0.0) if relu else o

  return pl.pallas_call(
      body,
      grid=(grid,),
      in_specs=[
          pl.BlockSpec((blk, d_in), lambda i: (i, 0)),
          pl.BlockSpec((blk, d_in), lambda i: (i, 0)),
          pl.BlockSpec((blk, d_in), lambda i: (i, 0)),
          pl.BlockSpec((blk, _NW), lambda i: (i, 0)),
          pl.BlockSpec((d_in, d_out), lambda i: (0, 0)),
          pl.BlockSpec((d_in, d_out), lambda i: (0, 0)),
          pl.BlockSpec((1, d_out), lambda i: (0, 0)),
      ],
      out_specs=pl.BlockSpec((blk, d_out), lambda i: (i, 0)),
      out_shape=jax.ShapeDtypeStruct((n, d_out), jnp.float32),
  )(h, a0, a1, dp, w_self, w_neigh, bias)


def kernel(x, W_self_0, W_neigh_0, b_0, W_self_1, W_neigh_1, b_1,
           W_self_2, W_neigh_2, b_2, edge_index_0, edge_index_1, edge_index_2):
  n, d = x.shape
  e = edge_index_0.shape[1]
  n_pad = ((n + 8 + 16 * _NS * 8 - 1) // (16 * _NS * 8)) * (16 * _NS * 8)
  step = _NW * _CHUNK * 4
  e_pad = ((e + step - 1) // step) * step
  n_chunks = e_pad // (_NW * _CHUNK)
  rows_per_sub = n_pad // _NS

  z2 = jnp.zeros((rows_per_sub, d), jnp.float32)
  pad_e = e_pad - e
  agg_fn = _sc_agg(n_pad, d, n_chunks)

  def seg_sum_parts(h_tab, edge_index):
    def lay(v, fill):
      v = jnp.concatenate([v, jnp.full((pad_e,), fill, jnp.int32)])
      v = v.reshape(_NW, n_chunks, 1, _CHUNK)
      return jnp.concatenate(
          [v, jnp.zeros((_NW, 2, 1, _CHUNK), jnp.int32)], axis=1)

    acc, degp = agg_fn(lay(edge_index[0], 0), lay(edge_index[1], n), h_tab, z2)
    dp = jnp.transpose(degp.reshape(_NW, n_pad))[:n]
    return acc[0, :n], acc[1, :n], dp

  a0, a1, dp = seg_sum_parts(x, edge_index_0)
  h = _combine(x, a0, a1, dp, W_self_0, W_neigh_0, b_0.reshape(1, -1), relu=True)
  a0, a1, dp = seg_sum_parts(h, edge_index_1)
  h = _combine(h, a0, a1, dp, W_self_1, W_neigh_1, b_1.reshape(1, -1), relu=True)
  d_out = W_self_2.shape[1]
  ws2 = jnp.pad(W_self_2, ((0, 0), (0, d - d_out)))
  wn2 = jnp.pad(W_neigh_2, ((0, 0), (0, d - d_out)))
  b2 = jnp.pad(b_2, (0, d - d_out)).reshape(1, -1)
  a0, a1, dp = seg_sum_parts(h, edge_index_2)
  out = _combine(h, a0, a1, dp, ws2, wn2, b2, relu=False)
  return out[:, :d_out]


# speedup vs baseline: 2.3336x; 2.3336x over previous
import functools

import jax
import jax.numpy as jnp
from jax import lax
from jax.experimental import pallas as pl
from jax.experimental.pallas import tpu as pltpu
from jax.experimental.pallas import tpu_sc as plsc

_NC = 2
_NS = 16
_NW = _NC * _NS
_L = 16
_CHUNK = 128


def _sc_agg(n_pad, width, n_chunks):
  rows_per_sub = n_pad // _NS
  mesh = plsc.VectorSubcoreMesh(core_axis_name="c", subcore_axis_name="s")

  @functools.partial(
      pl.kernel,
      out_type=(
          jax.ShapeDtypeStruct((_NC, n_pad, width), jnp.float32),
          jax.ShapeDtypeStruct((_NW * n_pad,), jnp.float32),
      ),
      mesh=mesh,
      compiler_params=pltpu.CompilerParams(needs_layout_passes=False,
                                           disable_bounds_checks=True),
      scratch_types=[
          pltpu.VMEM((n_chunks, _CHUNK), jnp.int32),
          pltpu.VMEM((n_chunks, _CHUNK), jnp.int32),
          pltpu.VMEM((_CHUNK, width), jnp.float32),
          pltpu.VMEM((n_pad,), jnp.float32),
          pltpu.VMEM_SHARED((n_pad, width), jnp.float32),
          pltpu.SemaphoreType.DMA,
          pltpu.SemaphoreType.DMA,
      ],
  )
  def k(src_hbm, dst_hbm, tab_hbm, z2_hbm,
        out_acc, out_degp, sidx_v, didx_v, rows_v, deg_v, acc_s, sem, isem):
    c = lax.axis_index("c")
    s = lax.axis_index("s")
    wid = s * _NC + c
    row0 = s * rows_per_sub

    pltpu.async_copy(src_hbm.at[wid], sidx_v, isem)
    pltpu.async_copy(dst_hbm.at[wid], didx_v, isem)
    pltpu.sync_copy(z2_hbm, acc_s.at[pl.ds(row0, rows_per_sub)])

    def zbody(i, carry):
      deg_v[pl.ds(i * _L, _L)] = jnp.zeros((_L,), jnp.float32)
      return carry

    lax.fori_loop(0, n_pad // _L, zbody, 0)
    pltpu.make_async_copy(src_hbm.at[0], sidx_v, isem).wait()
    pltpu.make_async_copy(dst_hbm.at[0], didx_v, isem).wait()
    plsc.subcore_barrier()

    ones = jnp.ones((_L,), jnp.float32)

    def body(i, carry):
      cp = pltpu.async_copy(tab_hbm.at[sidx_v.at[i]], rows_v, sem)
      for kk in range(_CHUNK // _L):
        idx = didx_v[i, pl.ds(kk * _L, _L)]
        plsc.addupdate_scatter(deg_v, [idx], ones)
      cp.wait()
      pltpu.sync_copy(rows_v, acc_s.at[didx_v.at[i]], add=True)
      return carry

    lax.fori_loop(0, n_chunks, body, 0)
    plsc.subcore_barrier()

    pltpu.sync_copy(acc_s.at[pl.ds(row0, rows_per_sub)],
                    out_acc.at[c, pl.ds(row0, rows_per_sub)])
    pltpu.sync_copy(deg_v, out_degp.at[pl.ds(wid * n_pad, n_pad)])

  return k


def _combine(h, a0, a1, dp, w_self, w_neigh, bias, relu):
  n, d_in = h.shape
  d_out = w_self.shape[1]
  blk = 1000
  grid = n // blk

  def body(h_r, a0_r, a1_r, dp_r, ws_r, wn_r, b_r, o_r):
    deg = jnp.sum(dp_r[...], axis=1, keepdims=True)
    r = 1.0 / jnp.maximum(deg, 1.0)
    agg = (a0_r[...] + a1_r[...]) * r
    o = (jnp.dot(h_r[...], ws_r[...], preferred_element_type=jnp.float32)
         + jnp.dot(agg, wn_r[...], preferred_element_type=jnp.float32)
         + b_r[...])
    o_r[...] = jnp.maximum(o, 0.0) if relu else o

  return pl.pallas_call(
      body,
      grid=(grid,),
      in_specs=[
          pl.BlockSpec((blk, d_in), lambda i: (i, 0)),
          pl.BlockSpec((blk, d_in), lambda i: (i, 0)),
          pl.BlockSpec((blk, d_in), lambda i: (i, 0)),
          pl.BlockSpec((blk, _NW), lambda i: (i, 0)),
          pl.BlockSpec((d_in, d_out), lambda i: (0, 0)),
          pl.BlockSpec((d_in, d_out), lambda i: (0, 0)),
          pl.BlockSpec((1, d_out), lambda i: (0, 0)),
      ],
      out_specs=pl.BlockSpec((blk, d_out), lambda i: (i, 0)),
      out_shape=jax.ShapeDtypeStruct((n, d_out), jnp.float32),
  )(h, a0, a1, dp, w_self, w_neigh, bias)


def kernel(x, W_self_0, W_neigh_0, b_0, W_self_1, W_neigh_1, b_1,
           W_self_2, W_neigh_2, b_2, edge_index_0, edge_index_1, edge_index_2):
  n, d = x.shape
  e = edge_index_0.shape[1]
  n_pad = ((n + 8 + 16 * _NS * 8 - 1) // (16 * _NS * 8)) * (16 * _NS * 8)
  step = _NW * _CHUNK
  e_pad = ((e + step - 1) // step) * step
  n_chunks = e_pad // step
  rows_per_sub = n_pad // _NS

  z2 = jnp.zeros((rows_per_sub, d), jnp.float32)
  pad_e = e_pad - e
  agg_fn = _sc_agg(n_pad, d, n_chunks)

  def seg_sum_parts(h_tab, edge_index):
    src = jnp.concatenate(
        [edge_index[0], jnp.zeros((pad_e,), jnp.int32)]).reshape(
            _NW, n_chunks, _CHUNK)
    dst = jnp.concatenate(
        [edge_index[1], jnp.full((pad_e,), n, jnp.int32)]).reshape(
            _NW, n_chunks, _CHUNK)
    acc, degp = agg_fn(src, dst, h_tab, z2)
    dp = jnp.transpose(degp.reshape(_NW, n_pad))[:n]
    return acc[0, :n], acc[1, :n], dp

  a0, a1, dp = seg_sum_parts(x, edge_index_0)
  h = _combine(x, a0, a1, dp, W_self_0, W_neigh_0, b_0.reshape(1, -1), relu=True)
  a0, a1, dp = seg_sum_parts(h, edge_index_1)
  h = _combine(h, a0, a1, dp, W_self_1, W_neigh_1, b_1.reshape(1, -1), relu=True)
  d_out = W_self_2.shape[1]
  ws2 = jnp.pad(W_self_2, ((0, 0), (0, d - d_out)))
  wn2 = jnp.pad(W_neigh_2, ((0, 0), (0, d - d_out)))
  b2 = jnp.pad(b_2, (0, d - d_out)).reshape(1, -1)
  a0, a1, dp = seg_sum_parts(h, edge_index_2)
  out = _combine(h, a0, a1, dp, ws2, wn2, b2, relu=False)
  return out[:, :d_out]
